# baseline (speedup 1.0000x reference)
import functools

import jax
import jax.numpy as jnp
from jax import lax
from jax.experimental import pallas as pl
from jax.experimental.pallas import tpu as pltpu
from jax.experimental.pallas import tpu_sc as plsc

H, W = 32, 32
N = H * W + 1
NUM_HEADS = 16
NUM_REL = (2 * H - 1) * (2 * W - 1) + 3
TABLE_FLAT = NUM_REL * NUM_HEADS

NC, NS, L = 2, 16, 16
NW = NC * NS
ROWS_PER_TILE = N // NW
R = 16
NCH = ROWS_PER_TILE // R
FULL_VPR = N // L
TAIL_POS = N - L


def _sc_gather(table_flat, idx):
    mesh = plsc.VectorSubcoreMesh(
        core_axis_name="c", subcore_axis_name="s", num_cores=NC, num_subcores=NS
    )

    @functools.partial(
        pl.kernel,
        out_type=jax.ShapeDtypeStruct((NUM_HEADS, N, N), jnp.float32),
        mesh=mesh,
        compiler_params=pltpu.CompilerParams(
            use_tc_tiling_on_sc=False, needs_layout_passes=False
        ),
        scratch_types=[
            pltpu.VMEM((TABLE_FLAT,), jnp.float32),
            pltpu.VMEM((2, R, N), jnp.int32),
            pltpu.VMEM((2, R, N), jnp.float32),
            pltpu.SemaphoreType.DMA,
            pltpu.SemaphoreType.DMA,
        ],
    )
    def k(table_hbm, idx_hbm, out_hbm, table_v, idx_v, out_v, sem_i, sem_o):
        wid = lax.axis_index("s") * NC + lax.axis_index("c")
        pltpu.sync_copy(table_hbm, table_v)
        tail_cols = lax.iota(jnp.int32, L) + TAIL_POS

        def idx_start(slot, r0):
            pltpu.async_copy(idx_hbm.at[pl.ds(r0, R), :], idx_v.at[slot], sem_i)

        def idx_wait():
            pltpu.make_async_copy(
                idx_hbm.at[pl.ds(0, R), :], idx_v.at[0], sem_i
            ).wait()

        def out_start(slot, h, r0):
            pltpu.async_copy(
                out_v.at[slot], out_hbm.at[h, pl.ds(r0, R), :], sem_o
            )

        def out_wait():
            pltpu.make_async_copy(
                out_v.at[0], out_hbm.at[0, pl.ds(0, R), :], sem_o
            ).wait()

        def fill(islot, oslot, h):
            def body(t, _):
                r = t >> 6
                pos = pl.multiple_of((t & 63) * L, L)
                iv = idx_v[islot, r, pl.ds(pos, L)]
                out_v[oslot, r, pl.ds(pos, L)] = plsc.load_gather(
                    table_v, [iv * NUM_HEADS + h]
                )
                return 0

            lax.fori_loop(0, R * FULL_VPR, body, 0, unroll=4)

            def tail(r, _):
                rvec = jnp.full((L,), r, jnp.int32)
                iv = plsc.load_gather(idx_v.at[islot], [rvec, tail_cols])
                vals = plsc.load_gather(table_v, [iv * NUM_HEADS + h])
                plsc.store_scatter(out_v.at[oslot], [rvec, tail_cols], vals)
                return 0

            lax.fori_loop(0, R, tail, 0, unroll=2)

        base = wid * ROWS_PER_TILE
        idx_start(0, base)
        for c in range(NCH):
            r0 = base + c * R
            if c + 1 < NCH:
                idx_start(c + 1, base + (c + 1) * R)
            idx_wait()
            for h in range(NUM_HEADS):
                step = c * NUM_HEADS + h
                if step >= 2:
                    out_wait()
                fill(c % 2, step % 2, h)
                out_start(step % 2, h, r0)

        @pl.when(wid == NW - 1)
        def _():
            idx_start(0, N - R)
            idx_wait()
            for h in range(NUM_HEADS):
                out_wait()
                fill(0, h % 2, h)
                out_start(h % 2, h, N - R)

        out_wait()
        out_wait()

    return k(table_flat, idx)


def kernel(relative_position_bias_table, relative_position_index):
    table_flat = relative_position_bias_table.reshape(-1)
    idx = relative_position_index.astype(jnp.int32)
    out = _sc_gather(table_flat, idx)
    return out.reshape(1, NUM_HEADS, N, N)


# speedup vs baseline: 1.4938x; 1.4938x over previous
import functools

import jax
import jax.numpy as jnp
from jax import lax
from jax.experimental import pallas as pl
from jax.experimental.pallas import tpu as pltpu
from jax.experimental.pallas import tpu_sc as plsc

H, W = 32, 32
N = H * W + 1
NUM_HEADS = 16
NUM_REL = (2 * H - 1) * (2 * W - 1) + 3
TABLE_FLAT = NUM_REL * NUM_HEADS

NC, NS, L = 2, 16, 16
NW = NC * NS
ROWS_PER_TILE = N // NW
R = 16
NCH = ROWS_PER_TILE // R
FULL_VPR = N // L
TAIL_POS = N - L


def _sc_gather(table_flat, idx):
    mesh = plsc.VectorSubcoreMesh(
        core_axis_name="c", subcore_axis_name="s", num_cores=NC, num_subcores=NS
    )

    @functools.partial(
        pl.kernel,
        out_type=jax.ShapeDtypeStruct((NUM_HEADS, N, N), jnp.float32),
        mesh=mesh,
        compiler_params=pltpu.CompilerParams(
            use_tc_tiling_on_sc=False, needs_layout_passes=False
        ),
        scratch_types=[
            pltpu.VMEM((TABLE_FLAT,), jnp.float32),
            pltpu.VMEM((2, R, N), jnp.int32),
            pltpu.VMEM((2, R, N), jnp.float32),
            pltpu.SemaphoreType.DMA,
            pltpu.SemaphoreType.DMA,
        ],
    )
    def k(table_hbm, idx_hbm, out_hbm, table_v, idx_v, out_v, sem_i, sem_o):
        wid = lax.axis_index("s") * NC + lax.axis_index("c")
        pltpu.sync_copy(table_hbm, table_v)
        tail_cols = lax.iota(jnp.int32, L) + TAIL_POS

        def idx_start(slot, r0):
            pltpu.async_copy(idx_hbm.at[pl.ds(r0, R), :], idx_v.at[slot], sem_i)

        def idx_wait():
            pltpu.make_async_copy(
                idx_hbm.at[pl.ds(0, R), :], idx_v.at[0], sem_i
            ).wait()

        def out_start(slot, h, r0):
            pltpu.async_copy(
                out_v.at[slot], out_hbm.at[h, pl.ds(r0, R), :], sem_o
            )

        def out_wait():
            pltpu.make_async_copy(
                out_v.at[0], out_hbm.at[0, pl.ds(0, R), :], sem_o
            ).wait()

        def fill(islot, oslot, h):
            @plsc.parallel_loop(0, R * FULL_VPR, unroll=8)
            def _(t):
                r = t >> 6
                pos = pl.multiple_of((t & 63) * L, L)
                iv = idx_v[islot, r, pl.ds(pos, L)]
                out_v[oslot, r, pl.ds(pos, L)] = plsc.load_gather(
                    table_v, [iv * NUM_HEADS + h]
                )

            @plsc.parallel_loop(0, R, unroll=4)
            def _(r):
                rvec = jnp.full((L,), r, jnp.int32)
                iv = plsc.load_gather(idx_v.at[islot], [rvec, tail_cols])
                vals = plsc.load_gather(table_v, [iv * NUM_HEADS + h])
                plsc.store_scatter(out_v.at[oslot], [rvec, tail_cols], vals)

        def head_loop(h_lo, islot, r0):
            def body(h, _):
                out_wait()
                fill(islot, h & 1, h)
                out_start(h & 1, h, r0)
                return 0

            lax.fori_loop(h_lo, NUM_HEADS, body, 0)

        base = wid * ROWS_PER_TILE
        idx_start(0, base)
        idx_start(1, base + R)
        idx_wait()
        fill(0, 0, 0)
        out_start(0, 0, base)
        fill(0, 1, 1)
        out_start(1, 1, base)
        head_loop(2, 0, base)
        for c in range(1, NCH):
            idx_wait()
            head_loop(0, c % 2, base + c * R)

        @pl.when(wid == NW - 1)
        def _():
            idx_start(0, N - R)
            idx_wait()
            head_loop(0, 0, N - R)

        out_wait()
        out_wait()

    return k(table_flat, idx)


def kernel(relative_position_bias_table, relative_position_index):
    table_flat = relative_position_bias_table.reshape(-1)
    idx = relative_position_index.astype(jnp.int32)
    out = _sc_gather(table_flat, idx)
    return out.reshape(1, NUM_HEADS, N, N)


# speedup vs baseline: 2.0302x; 1.3590x over previous
import functools

import jax
import jax.numpy as jnp
from jax import lax
from jax.experimental import pallas as pl
from jax.experimental.pallas import tpu as pltpu
from jax.experimental.pallas import tpu_sc as plsc

H, W = 32, 32
N = H * W + 1
NUM_HEADS = 16
NUM_REL = (2 * H - 1) * (2 * W - 1) + 3
TABLE_FLAT = NUM_REL * NUM_HEADS

NC, NS, L = 2, 16, 16
NW = NC * NS
ROWS_PER_TILE = N // NW
R = 16
NCH = ROWS_PER_TILE // R
FULL_VPR = N // L
TAIL_POS = N - L


def _sc_gather(table_flat, idx):
    mesh = plsc.VectorSubcoreMesh(
        core_axis_name="c", subcore_axis_name="s", num_cores=NC, num_subcores=NS
    )

    @functools.partial(
        pl.kernel,
        out_type=jax.ShapeDtypeStruct((NUM_HEADS, N, N), jnp.float32),
        mesh=mesh,
        compiler_params=pltpu.CompilerParams(
            use_tc_tiling_on_sc=False, needs_layout_passes=False
        ),
        scratch_types=[
            pltpu.VMEM((TABLE_FLAT,), jnp.float32),
            pltpu.VMEM((2, R, N), jnp.int32),
            pltpu.VMEM((2, R, N), jnp.float32),
            pltpu.SemaphoreType.DMA,
            pltpu.SemaphoreType.DMA,
        ],
    )
    def k(table_hbm, idx_hbm, out_hbm, table_v, idx_v, out_v, sem_i, sem_o):
        wid = lax.axis_index("s") * NC + lax.axis_index("c")
        pltpu.sync_copy(table_hbm, table_v)
        tail_cols = lax.iota(jnp.int32, L) + TAIL_POS

        def idx_start(slot, r0):
            pltpu.async_copy(idx_hbm.at[pl.ds(r0, R), :], idx_v.at[slot], sem_i)

        def idx_wait():
            pltpu.make_async_copy(
                idx_hbm.at[pl.ds(0, R), :], idx_v.at[0], sem_i
            ).wait()

        def out_start(slot, h, r0):
            pltpu.async_copy(
                out_v.at[slot], out_hbm.at[h, pl.ds(r0, R), :], sem_o
            )

        def out_wait():
            pltpu.make_async_copy(
                out_v.at[0], out_hbm.at[0, pl.ds(0, R), :], sem_o
            ).wait()

        def fill(islot, oslot, h):
            @plsc.parallel_loop(0, R * FULL_VPR, unroll=8)
            def _(t):
                r = t >> 6
                pos = pl.multiple_of((t & 63) * L, L)
                iv = idx_v[islot, r, pl.ds(pos, L)]
                out_v[oslot, r, pl.ds(pos, L)] = plsc.load_gather(
                    table_v, [iv + h * NUM_REL]
                )

            @plsc.parallel_loop(0, R, unroll=4)
            def _(r):
                rvec = jnp.full((L,), r, jnp.int32)
                iv = plsc.load_gather(idx_v.at[islot], [rvec, tail_cols])
                vals = plsc.load_gather(table_v, [iv + h * NUM_REL])
                plsc.store_scatter(out_v.at[oslot], [rvec, tail_cols], vals)

        def head_loop(h_lo, islot, r0):
            def body(h, _):
                out_wait()
                fill(islot, h & 1, h)
                out_start(h & 1, h, r0)
                return 0

            lax.fori_loop(h_lo, NUM_HEADS, body, 0)

        base = wid * ROWS_PER_TILE
        idx_start(0, base)
        idx_start(1, base + R)
        idx_wait()
        fill(0, 0, 0)
        out_start(0, 0, base)
        fill(0, 1, 1)
        out_start(1, 1, base)
        head_loop(2, 0, base)
        for c in range(1, NCH):
            idx_wait()
            head_loop(0, c % 2, base + c * R)

        @pl.when(wid == NW - 1)
        def _():
            idx_start(0, N - R)
            idx_wait()
            head_loop(0, 0, N - R)

        out_wait()
        out_wait()

    return k(table_flat, idx)


def kernel(relative_position_bias_table, relative_position_index):
    table_flat = relative_position_bias_table.T.reshape(-1)
    idx = relative_position_index.astype(jnp.int32)
    out = _sc_gather(table_flat, idx)
    return out.reshape(1, NUM_HEADS, N, N)


# speedup vs baseline: 2.0955x; 1.0322x over previous
import functools

import jax
import jax.numpy as jnp
from jax import lax
from jax.experimental import pallas as pl
from jax.experimental.pallas import tpu as pltpu
from jax.experimental.pallas import tpu_sc as plsc

H, W = 32, 32
N = H * W + 1
NUM_HEADS = 16
NUM_REL = (2 * H - 1) * (2 * W - 1) + 3
TABLE_FLAT = NUM_REL * NUM_HEADS

NC, NS, L = 2, 16, 16
NW = NC * NS
ROWS_PER_TILE = N // NW
R = 8
NCH = ROWS_PER_TILE // R
NPAIR = NUM_HEADS // 2
FULL_VPR = N // L
TAIL_POS = N - L


def _sc_gather(table_flat, idx):
    mesh = plsc.VectorSubcoreMesh(
        core_axis_name="c", subcore_axis_name="s", num_cores=NC, num_subcores=NS
    )

    @functools.partial(
        pl.kernel,
        out_type=jax.ShapeDtypeStruct((NUM_HEADS, N, N), jnp.float32),
        mesh=mesh,
        compiler_params=pltpu.CompilerParams(
            use_tc_tiling_on_sc=False, needs_layout_passes=False
        ),
        scratch_types=[
            pltpu.VMEM((TABLE_FLAT,), jnp.float32),
            pltpu.VMEM((2, R, N), jnp.int32),
            pltpu.VMEM((4, R, N), jnp.float32),
            pltpu.SemaphoreType.DMA,
            pltpu.SemaphoreType.DMA,
        ],
    )
    def k(table_hbm, idx_hbm, out_hbm, table_v, idx_v, out_v, sem_i, sem_o):
        wid = lax.axis_index("s") * NC + lax.axis_index("c")
        pltpu.sync_copy(table_hbm, table_v)
        tail_cols = lax.iota(jnp.int32, L) + TAIL_POS

        def idx_start(slot, r0):
            pltpu.async_copy(idx_hbm.at[pl.ds(r0, R), :], idx_v.at[slot], sem_i)

        def idx_wait():
            pltpu.make_async_copy(
                idx_hbm.at[pl.ds(0, R), :], idx_v.at[0], sem_i
            ).wait()

        def out_start(slot, h, r0):
            pltpu.async_copy(
                out_v.at[slot], out_hbm.at[h, pl.ds(r0, R), :], sem_o
            )

        def out_wait():
            pltpu.make_async_copy(
                out_v.at[0], out_hbm.at[0, pl.ds(0, R), :], sem_o
            ).wait()

        def fill_pair(islot, oslot, h0):
            b0 = h0 * NUM_REL

            @plsc.parallel_loop(0, R * FULL_VPR, unroll=4)
            def _(t):
                r = t >> 6
                pos = pl.multiple_of((t & 63) * L, L)
                iv = idx_v[islot, r, pl.ds(pos, L)] + b0
                out_v[oslot, r, pl.ds(pos, L)] = plsc.load_gather(
                    table_v, [iv]
                )
                out_v[oslot + 1, r, pl.ds(pos, L)] = plsc.load_gather(
                    table_v, [iv + NUM_REL]
                )

            @plsc.parallel_loop(0, R, unroll=2)
            def _(r):
                rvec = jnp.full((L,), r, jnp.int32)
                iv = plsc.load_gather(idx_v.at[islot], [rvec, tail_cols]) + b0
                vals0 = plsc.load_gather(table_v, [iv])
                vals1 = plsc.load_gather(table_v, [iv + NUM_REL])
                plsc.store_scatter(out_v.at[oslot], [rvec, tail_cols], vals0)
                plsc.store_scatter(
                    out_v.at[oslot + 1], [rvec, tail_cols], vals1
                )

        def pair_loop(p_lo, islot, r0):
            def body(p, _):
                out_wait()
                out_wait()
                slot = (p & 1) * 2
                fill_pair(islot, slot, p * 2)
                out_start(slot, p * 2, r0)
                out_start(slot + 1, p * 2 + 1, r0)
                return 0

            lax.fori_loop(p_lo, NPAIR, body, 0)

        base = wid * ROWS_PER_TILE
        idx_start(0, base)
        idx_start(1, base + R)
        idx_wait()
        fill_pair(0, 0, 0)
        out_start(0, 0, base)
        out_start(1, 1, base)
        fill_pair(0, 2, 2)
        out_start(2, 2, base)
        out_start(3, 3, base)
        pair_loop(2, 0, base)
        for c in range(1, NCH):
            if c + 1 < NCH:
                idx_start((c + 1) % 2, base + (c + 1) * R)
            idx_wait()
            pair_loop(0, c % 2, base + c * R)

        @pl.when(wid == NW - 1)
        def _():
            idx_start(0, N - R)
            idx_wait()
            pair_loop(0, 0, N - R)

        out_wait()
        out_wait()
        out_wait()
        out_wait()

    return k(table_flat, idx)


def kernel(relative_position_bias_table, relative_position_index):
    table_flat = relative_position_bias_table.T.reshape(-1)
    idx = relative_position_index.astype(jnp.int32)
    out = _sc_gather(table_flat, idx)
    return out.reshape(1, NUM_HEADS, N, N)


# speedup vs baseline: 8.3971x; 4.0071x over previous
import functools

import jax
import jax.numpy as jnp
from jax import lax
from jax.experimental import pallas as pl
from jax.experimental.pallas import tpu as pltpu
from jax.experimental.pallas import tpu_sc as plsc

H, W = 32, 32
N = H * W + 1
NUM_HEADS = 16
NUM_REL = (2 * H - 1) * (2 * W - 1) + 3
TABLE_FLAT = NUM_REL * NUM_HEADS

NC, NS, L = 2, 16, 16
NW = NC * NS
ROWS_PER_TILE = (N - 1) // NW
R = 8
NCH = ROWS_PER_TILE // R
FULL_VPR = (N - 1) // L


def _sc_gather(table_flat, idx):
    mesh = plsc.VectorSubcoreMesh(
        core_axis_name="c", subcore_axis_name="s", num_cores=NC, num_subcores=NS
    )

    @functools.partial(
        pl.kernel,
        out_type=jax.ShapeDtypeStruct((NUM_HEADS, N, N), jnp.float32),
        mesh=mesh,
        compiler_params=pltpu.CompilerParams(needs_layout_passes=False),
        scratch_types=[
            pltpu.VMEM((TABLE_FLAT,), jnp.float32),
            pltpu.VMEM((2, R, N), jnp.int32),
            pltpu.VMEM((2, R, N), jnp.float32),
            pltpu.VMEM((1, N), jnp.int32),
            pltpu.VMEM((2, 1, N), jnp.float32),
            pltpu.SemaphoreType.DMA,
            pltpu.SemaphoreType.DMA,
        ],
    )
    def k(table_hbm, idx_hbm, out_hbm, table_v, idx_v, out_v, idxrow_v,
          outrow_v, sem_i, sem_o):
        wid = lax.axis_index("s") * NC + lax.axis_index("c")
        pltpu.sync_copy(table_hbm, table_v)
        rows_iota = lax.iota(jnp.int32, L)
        col_last = jnp.full((L,), N - 1, jnp.int32)
        lane0 = rows_iota == 0
        chunk_lanes = rows_iota < R

        def idx_start(slot, r0):
            pltpu.async_copy(idx_hbm.at[pl.ds(r0, R), :], idx_v.at[slot], sem_i)

        def idx_wait():
            pltpu.make_async_copy(
                idx_hbm.at[pl.ds(0, R), :], idx_v.at[0], sem_i
            ).wait()

        def out_start(slot, h, r0):
            pltpu.async_copy(
                out_v.at[slot], out_hbm.at[h, pl.ds(r0, R), :], sem_o
            )

        def out_wait():
            pltpu.make_async_copy(
                out_v.at[0], out_hbm.at[0, pl.ds(0, R), :], sem_o
            ).wait()

        def fill(islot, oslot, h):
            hbase = h * NUM_REL

            @plsc.parallel_loop(0, R * FULL_VPR, unroll=8)
            def _(t):
                r = t >> 6
                pos = pl.multiple_of((t & 63) * L, L)
                iv = idx_v[islot, r, pl.ds(pos, L)]
                out_v[oslot, r, pl.ds(pos, L)] = plsc.load_gather(
                    table_v, [iv + hbase]
                )

            iv = plsc.load_gather(
                idx_v.at[islot], [rows_iota, col_last], mask=chunk_lanes
            )
            vals = plsc.load_gather(table_v, [iv + hbase], mask=chunk_lanes)
            plsc.store_scatter(
                out_v.at[oslot], [rows_iota, col_last], vals, mask=chunk_lanes
            )

        def head_loop(h_lo, islot, r0):
            def body(h, _):
                out_wait()
                fill(islot, h & 1, h)
                out_start(h & 1, h, r0)
                return 0

            lax.fori_loop(h_lo, NUM_HEADS, body, 0)

        base = wid * ROWS_PER_TILE
        idx_start(0, base)
        idx_start(1, base + R)
        idx_wait()
        fill(0, 0, 0)
        out_start(0, 0, base)
        fill(0, 1, 1)
        out_start(1, 1, base)
        head_loop(2, 0, base)
        for c in range(1, NCH):
            if c + 1 < NCH:
                idx_start((c + 1) % 2, base + (c + 1) * R)
            idx_wait()
            head_loop(0, c % 2, base + c * R)

        out_wait()
        out_wait()

        @pl.when(wid == NW - 1)
        def _():
            pltpu.sync_copy(idx_hbm.at[pl.ds(N - 1, 1), :], idxrow_v)

            def rowfill(h, oslot):
                hbase = h * NUM_REL

                @plsc.parallel_loop(0, FULL_VPR, unroll=8)
                def _(t):
                    pos = pl.multiple_of(t * L, L)
                    iv = idxrow_v[0, pl.ds(pos, L)]
                    outrow_v[oslot, 0, pl.ds(pos, L)] = plsc.load_gather(
                        table_v, [iv + hbase]
                    )

                zeros = jnp.zeros((L,), jnp.int32)
                iv = plsc.load_gather(idxrow_v, [zeros, col_last], mask=lane0)
                vals = plsc.load_gather(table_v, [iv + hbase], mask=lane0)
                plsc.store_scatter(
                    outrow_v.at[oslot], [zeros, col_last], vals, mask=lane0
                )
                pltpu.async_copy(
                    outrow_v.at[oslot], out_hbm.at[h, pl.ds(N - 1, 1), :],
                    sem_o,
                )

            def row_wait():
                pltpu.make_async_copy(
                    outrow_v.at[0], out_hbm.at[0, pl.ds(0, 1), :], sem_o
                ).wait()

            rowfill(0, 0)
            rowfill(1, 1)

            def body(h, _):
                row_wait()
                rowfill(h, h & 1)
                return 0

            lax.fori_loop(2, NUM_HEADS, body, 0)
            row_wait()
            row_wait()

    return k(table_flat, idx)


def kernel(relative_position_bias_table, relative_position_index):
    table_flat = relative_position_bias_table.T.reshape(-1)
    idx = relative_position_index.astype(jnp.int32)
    out = _sc_gather(table_flat, idx)
    return out.reshape(1, NUM_HEADS, N, N)
